# baseline (speedup 1.0000x reference)
import functools

import jax
import jax.numpy as jnp
from jax import lax
from jax.experimental import pallas as pl
from jax.experimental.pallas import tpu as pltpu
from jax.experimental.pallas import tpu_sc as plsc

_HIDDEN = 1024
_BATCH = 4
_SEQ = 2048
_TOK = _BATCH * _SEQ
_NW = 32
_SPW = _SEQ // _NW
_CH = 8
_NG = _SPW // _CH
_LANES = 16
_HG = _HIDDEN // _LANES

_mesh = plsc.VectorSubcoreMesh(core_axis_name="c", subcore_axis_name="s")


@functools.partial(
    pl.kernel,
    out_type=jax.ShapeDtypeStruct((_TOK, _HIDDEN), jnp.float32),
    mesh=_mesh,
    scratch_types=[
        pltpu.VMEM((_BATCH * _SPW,), jnp.int32),
        [pltpu.VMEM((_CH, _HIDDEN), jnp.float32)] * (2 * _BATCH),
        [pltpu.VMEM((_CH, _HIDDEN), jnp.float32)] * 2,
        [pltpu.SemaphoreType.DMA] * (2 * _BATCH),
        [pltpu.SemaphoreType.DMA] * (2 * _BATCH),
        [pltpu.SemaphoreType.DMA] * 2,
    ],
)
def _emb_kernel(idx_hbm, wtab_hbm, ptab_hbm, out_hbm, idx_v, wbufs, pbufs,
                gsems, osems, psems):
    wid = lax.axis_index("s") * 2 + lax.axis_index("c")
    s0 = wid * _SPW

    idx_descs = [
        pltpu.async_copy(idx_hbm.at[pl.ds(b * _SEQ + s0, _SPW)],
                         idx_v.at[pl.ds(b * _SPW, _SPW)], osems[b])
        for b in range(_BATCH)
    ]
    for d in idx_descs:
        d.wait()

    def gather(j, b):
        slot = (j % 2) * _BATCH + b
        return pltpu.async_copy(
            wtab_hbm.at[idx_v.at[pl.ds(b * _SPW + j * _CH, _CH)]],
            wbufs[slot], gsems[slot])

    def writeback(j, b):
        slot = (j % 2) * _BATCH + b
        row0 = b * _SEQ + s0 + j * _CH
        return pltpu.async_copy(wbufs[slot], out_hbm.at[pl.ds(row0, _CH)],
                                osems[slot])

    def load_pos(j):
        return pltpu.async_copy(ptab_hbm.at[pl.ds(s0 + j * _CH, _CH)],
                                pbufs[j % 2], psems[j % 2])

    p_descs = [None] * _NG
    g_descs = [[None] * _BATCH for _ in range(_NG)]
    o_descs = [[None] * _BATCH for _ in range(_NG)]
    p_descs[0] = load_pos(0)
    for b in range(_BATCH):
        g_descs[0][b] = gather(0, b)

    for j in range(_NG):
        half = j % 2
        if j + 1 < _NG:
            p_descs[j + 1] = load_pos(j + 1)
            if j >= 1:
                for b in range(_BATCH):
                    o_descs[j - 1][b].wait()
            for b in range(_BATCH):
                g_descs[j + 1][b] = gather(j + 1, b)
        p_descs[j].wait()
        for b in range(_BATCH):
            g_descs[j][b].wait()
        pb = pbufs[half]
        for pair in range(2):
            wset = wbufs[half * _BATCH + 2 * pair:half * _BATCH + 2 * pair + 2]

            def row(r, _, wset=wset, pb=pb):
                for g in range(_HG):
                    sl = pl.ds(g * _LANES, _LANES)
                    x = pb[r, sl]
                    for wb in wset:
                        plsc.addupdate(wb.at[r, sl], x)
                return 0

            lax.fori_loop(0, _CH, row, 0)
            for b in (2 * pair, 2 * pair + 1):
                o_descs[j][b] = writeback(j, b)

    for b in range(_BATCH):
        o_descs[_NG - 2][b].wait()
        o_descs[_NG - 1][b].wait()


def kernel(inputs, word_embeddings, position_embeddings):
    flat_idx = inputs.reshape(_TOK).astype(jnp.int32)
    out = _emb_kernel(flat_idx, word_embeddings, position_embeddings)
    return out.reshape(_BATCH, _SEQ, _HIDDEN)


# speedup vs baseline: 1.0127x; 1.0127x over previous
import functools

import jax
import jax.numpy as jnp
from jax import lax
from jax.experimental import pallas as pl
from jax.experimental.pallas import tpu as pltpu
from jax.experimental.pallas import tpu_sc as plsc

_HIDDEN = 1024
_BATCH = 4
_SEQ = 2048
_TOK = _BATCH * _SEQ
_NW = 32
_SPW = _SEQ // _NW
_CH = 8
_NG = _SPW // _CH
_LANES = 16
_HG = _HIDDEN // _LANES

_mesh = plsc.VectorSubcoreMesh(core_axis_name="c", subcore_axis_name="s")


@functools.partial(
    pl.kernel,
    out_type=jax.ShapeDtypeStruct((_TOK, _HIDDEN), jnp.float32),
    mesh=_mesh,
    scratch_types=[
        pltpu.VMEM((_BATCH * _SPW,), jnp.int32),
        [pltpu.VMEM((_CH, _HIDDEN), jnp.float32)] * (2 * _BATCH),
        [pltpu.VMEM((_CH, _HIDDEN), jnp.float32)] * 2,
        [pltpu.SemaphoreType.DMA] * (2 * _BATCH),
        [pltpu.SemaphoreType.DMA] * (2 * _BATCH),
        [pltpu.SemaphoreType.DMA] * 2,
    ],
)
def _emb_kernel(idx_hbm, wtab_hbm, ptab_hbm, out_hbm, idx_v, wbufs, pbufs,
                gsems, osems, psems):
    wid = lax.axis_index("s") * 2 + lax.axis_index("c")
    s0 = wid * _SPW

    idx_descs = [
        pltpu.async_copy(idx_hbm.at[pl.ds(b * _SEQ + s0, _SPW)],
                         idx_v.at[pl.ds(b * _SPW, _SPW)], osems[b])
        for b in range(_BATCH)
    ]
    for d in idx_descs:
        d.wait()

    def gather(j, b):
        slot = (j % 2) * _BATCH + b
        return pltpu.async_copy(
            wtab_hbm.at[idx_v.at[pl.ds(b * _SPW + j * _CH, _CH)]],
            wbufs[slot], gsems[slot])

    def writeback(j, b):
        slot = (j % 2) * _BATCH + b
        row0 = b * _SEQ + s0 + j * _CH
        return pltpu.async_copy(wbufs[slot], out_hbm.at[pl.ds(row0, _CH)],
                                osems[slot])

    def load_pos(j):
        return pltpu.async_copy(ptab_hbm.at[pl.ds(s0 + j * _CH, _CH)],
                                pbufs[j % 2], psems[j % 2])

    p_descs = [None] * _NG
    g_descs = [[None] * _BATCH for _ in range(_NG)]
    o_descs = [[None] * _BATCH for _ in range(_NG)]
    p_descs[0] = load_pos(0)
    for b in range(_BATCH):
        g_descs[0][b] = gather(0, b)

    for j in range(_NG):
        half = j % 2
        if j + 1 < _NG:
            p_descs[j + 1] = load_pos(j + 1)
            if j >= 1:
                for b in range(_BATCH):
                    o_descs[j - 1][b].wait()
            for b in range(_BATCH):
                g_descs[j + 1][b] = gather(j + 1, b)
        p_descs[j].wait()
        for b in range(_BATCH):
            g_descs[j][b].wait()
        wset = wbufs[half * _BATCH:(half + 1) * _BATCH]
        pb = pbufs[half]

        def row(r, _, wset=wset, pb=pb):
            for g in range(_HG):
                sl = pl.ds(g * _LANES, _LANES)
                x = pb[r, sl]
                for wb in wset:
                    plsc.addupdate(wb.at[r, sl], x)
            return 0

        lax.fori_loop(0, _CH, row, 0)
        for b in range(_BATCH):
            o_descs[j][b] = writeback(j, b)

    for b in range(_BATCH):
        o_descs[_NG - 2][b].wait()
        o_descs[_NG - 1][b].wait()


def kernel(inputs, word_embeddings, position_embeddings):
    flat_idx = inputs.reshape(_TOK).astype(jnp.int32)
    out = _emb_kernel(flat_idx, word_embeddings, position_embeddings)
    return out.reshape(_BATCH, _SEQ, _HIDDEN)


# speedup vs baseline: 1.1466x; 1.1323x over previous
import functools

import jax
import jax.numpy as jnp
from jax import lax
from jax.experimental import pallas as pl
from jax.experimental.pallas import tpu as pltpu
from jax.experimental.pallas import tpu_sc as plsc

_HIDDEN = 1024
_BATCH = 4
_SEQ = 2048
_TOK = _BATCH * _SEQ
_NW = 32
_SPW = _SEQ // _NW
_CH = 8
_NG = _SPW // _CH
_LANES = 16
_HG = _HIDDEN // _LANES

_mesh = plsc.VectorSubcoreMesh(core_axis_name="c", subcore_axis_name="s")


@functools.partial(
    pl.kernel,
    out_type=jax.ShapeDtypeStruct((_TOK, _HIDDEN), jnp.float32),
    mesh=_mesh,
    scratch_types=[
        pltpu.VMEM((_BATCH * _SPW,), jnp.int32),
        [pltpu.VMEM((_CH, _HIDDEN), jnp.float32)] * (2 * _BATCH),
        [pltpu.VMEM((_CH, _HIDDEN), jnp.float32)] * 2,
        [pltpu.SemaphoreType.DMA] * (2 * _BATCH),
        [pltpu.SemaphoreType.DMA] * (2 * _BATCH),
        [pltpu.SemaphoreType.DMA] * 2,
    ],
)
def _emb_kernel(idx_hbm, wtab_hbm, ptab_hbm, out_hbm, idx_v, wbufs, pbufs,
                gsems, osems, psems):
    wid = lax.axis_index("s") * 2 + lax.axis_index("c")
    s0 = wid * _SPW

    idx_descs = [
        pltpu.async_copy(idx_hbm.at[pl.ds(b * _SEQ + s0, _SPW)],
                         idx_v.at[pl.ds(b * _SPW, _SPW)], osems[b])
        for b in range(_BATCH)
    ]
    for d in idx_descs:
        d.wait()

    def gather(j, b):
        slot = (j % 2) * _BATCH + b
        return pltpu.async_copy(
            wtab_hbm.at[idx_v.at[pl.ds(b * _SPW + j * _CH, _CH)]],
            wbufs[slot], gsems[slot])

    def writeback(j, b):
        slot = (j % 2) * _BATCH + b
        row0 = b * _SEQ + s0 + j * _CH
        return pltpu.async_copy(wbufs[slot], out_hbm.at[pl.ds(row0, _CH)],
                                osems[slot])

    def load_pos(j):
        return pltpu.async_copy(ptab_hbm.at[pl.ds(s0 + j * _CH, _CH)],
                                pbufs[j % 2], psems[j % 2])

    p_descs = [None] * _NG
    g_descs = [[None] * _BATCH for _ in range(_NG)]
    o_descs = [[None] * _BATCH for _ in range(_NG)]
    p_descs[0] = load_pos(0)
    for b in range(_BATCH):
        g_descs[0][b] = gather(0, b)

    for j in range(_NG):
        half = j % 2
        if j + 1 < _NG:
            p_descs[j + 1] = load_pos(j + 1)
            if j >= 1:
                for b in range(_BATCH):
                    o_descs[j - 1][b].wait()
            for b in range(_BATCH):
                g_descs[j + 1][b] = gather(j + 1, b)
        p_descs[j].wait()
        for b in range(_BATCH):
            g_descs[j][b].wait()
        wset = wbufs[half * _BATCH:(half + 1) * _BATCH]
        pb = pbufs[half]

        def blk(i, _, wset=wset, pb=pb):
            r = i // 4
            g0 = (i % 4) * 16
            for g in range(16):
                sl = pl.ds((g0 + g) * _LANES, _LANES)
                x = pb[r, sl]
                for wb in wset:
                    plsc.addupdate(wb.at[r, sl], x)
            return 0

        lax.fori_loop(0, _CH * 4, blk, 0)
        for b in range(_BATCH):
            o_descs[j][b] = writeback(j, b)

    for b in range(_BATCH):
        o_descs[_NG - 2][b].wait()
        o_descs[_NG - 1][b].wait()


def kernel(inputs, word_embeddings, position_embeddings):
    flat_idx = inputs.reshape(_TOK).astype(jnp.int32)
    out = _emb_kernel(flat_idx, word_embeddings, position_embeddings)
    return out.reshape(_BATCH, _SEQ, _HIDDEN)


# speedup vs baseline: 1.1650x; 1.0161x over previous
import functools

import jax
import jax.numpy as jnp
from jax import lax
from jax.experimental import pallas as pl
from jax.experimental.pallas import tpu as pltpu
from jax.experimental.pallas import tpu_sc as plsc

_HIDDEN = 1024
_BATCH = 4
_SEQ = 2048
_TOK = _BATCH * _SEQ
_NW = 32
_SPW = _SEQ // _NW
_CH = 8
_NG = _SPW // _CH
_LANES = 16
_HG = _HIDDEN // _LANES

_mesh = plsc.VectorSubcoreMesh(core_axis_name="c", subcore_axis_name="s")


@functools.partial(
    pl.kernel,
    out_type=jax.ShapeDtypeStruct((_TOK, _HIDDEN), jnp.float32),
    mesh=_mesh,
    scratch_types=[
        pltpu.VMEM((_BATCH * _SPW,), jnp.int32),
        [pltpu.VMEM((_CH, _HIDDEN), jnp.float32)] * (2 * _BATCH),
        [pltpu.VMEM((_CH, _HIDDEN), jnp.float32)] * 2,
        [pltpu.SemaphoreType.DMA] * (2 * _BATCH),
        [pltpu.SemaphoreType.DMA] * (2 * _BATCH),
        [pltpu.SemaphoreType.DMA] * 2,
    ],
)
def _emb_kernel(idx_hbm, wtab_hbm, ptab_hbm, out_hbm, idx_v, wbufs, pbufs,
                gsems, osems, psems):
    wid = lax.axis_index("s") * 2 + lax.axis_index("c")
    s0 = wid * _SPW

    idx_descs = [
        pltpu.async_copy(idx_hbm.at[pl.ds(b * _SEQ + s0, _SPW)],
                         idx_v.at[pl.ds(b * _SPW, _SPW)], osems[b])
        for b in range(_BATCH)
    ]
    for d in idx_descs:
        d.wait()

    def gather(j, b):
        slot = (j % 2) * _BATCH + b
        return pltpu.async_copy(
            wtab_hbm.at[idx_v.at[pl.ds(b * _SPW + j * _CH, _CH)]],
            wbufs[slot], gsems[slot])

    def writeback(j, b):
        slot = (j % 2) * _BATCH + b
        row0 = b * _SEQ + s0 + j * _CH
        return pltpu.async_copy(wbufs[slot], out_hbm.at[pl.ds(row0, _CH)],
                                osems[slot])

    def load_pos(j):
        return pltpu.async_copy(ptab_hbm.at[pl.ds(s0 + j * _CH, _CH)],
                                pbufs[j % 2], psems[j % 2])

    p_descs = [None] * _NG
    g_descs = [[None] * _BATCH for _ in range(_NG)]
    o_descs = [[None] * _BATCH for _ in range(_NG)]
    p_descs[0] = load_pos(0)
    for b in range(_BATCH):
        g_descs[0][b] = gather(0, b)

    for j in range(_NG):
        half = j % 2
        if j + 1 < _NG:
            p_descs[j + 1] = load_pos(j + 1)
            if j >= 1:
                for b in range(_BATCH):
                    o_descs[j - 1][b].wait()
            for b in range(_BATCH):
                g_descs[j + 1][b] = gather(j + 1, b)
        p_descs[j].wait()
        for b in range(_BATCH):
            g_descs[j][b].wait()
        wset = wbufs[half * _BATCH:(half + 1) * _BATCH]
        pb = pbufs[half]

        def blk(i, _, wset=wset, pb=pb):
            r = i // 8
            g0 = (i % 8) * 8
            for g in range(8):
                sl = pl.ds((g0 + g) * _LANES, _LANES)
                x = pb[r, sl]
                for wb in wset:
                    plsc.addupdate(wb.at[r, sl], x)
            return 0

        lax.fori_loop(0, _CH * 8, blk, 0)
        for b in range(_BATCH):
            o_descs[j][b] = writeback(j, b)

    for b in range(_BATCH):
        o_descs[_NG - 2][b].wait()
        o_descs[_NG - 1][b].wait()


def kernel(inputs, word_embeddings, position_embeddings):
    flat_idx = inputs.reshape(_TOK).astype(jnp.int32)
    out = _emb_kernel(flat_idx, word_embeddings, position_embeddings)
    return out.reshape(_BATCH, _SEQ, _HIDDEN)


# speedup vs baseline: 1.1798x; 1.0127x over previous
import functools

import jax
import jax.numpy as jnp
from jax import lax
from jax.experimental import pallas as pl
from jax.experimental.pallas import tpu as pltpu
from jax.experimental.pallas import tpu_sc as plsc

_HIDDEN = 1024
_BATCH = 4
_SEQ = 2048
_TOK = _BATCH * _SEQ
_NW = 32
_SPW = _SEQ // _NW
_CH = 8
_NG = _SPW // _CH
_LANES = 16
_HG = _HIDDEN // _LANES

_mesh = plsc.VectorSubcoreMesh(core_axis_name="c", subcore_axis_name="s")


@functools.partial(
    pl.kernel,
    out_type=jax.ShapeDtypeStruct((_TOK, _HIDDEN), jnp.float32),
    mesh=_mesh,
    scratch_types=[
        pltpu.VMEM((_BATCH * _SPW,), jnp.int32),
        [pltpu.VMEM((_CH, _HIDDEN), jnp.float32)] * (2 * _BATCH),
        [pltpu.VMEM((_CH, _HIDDEN), jnp.float32)] * 2,
        [pltpu.SemaphoreType.DMA] * (2 * _BATCH),
        [pltpu.SemaphoreType.DMA] * (2 * _BATCH),
        [pltpu.SemaphoreType.DMA] * 2,
    ],
)
def _emb_kernel(idx_hbm, wtab_hbm, ptab_hbm, out_hbm, idx_v, wbufs, pbufs,
                gsems, osems, psems):
    wid = lax.axis_index("s") * 2 + lax.axis_index("c")
    s0 = wid * _SPW

    idx_descs = [
        pltpu.async_copy(idx_hbm.at[pl.ds(b * _SEQ + s0, _SPW)],
                         idx_v.at[pl.ds(b * _SPW, _SPW)], osems[b])
        for b in range(_BATCH)
    ]
    for d in idx_descs:
        d.wait()

    def gather(j, b):
        slot = (j % 2) * _BATCH + b
        return pltpu.async_copy(
            wtab_hbm.at[idx_v.at[pl.ds(b * _SPW + j * _CH, _CH)]],
            wbufs[slot], gsems[slot])

    def writeback(j, b):
        slot = (j % 2) * _BATCH + b
        row0 = b * _SEQ + s0 + j * _CH
        return pltpu.async_copy(wbufs[slot], out_hbm.at[pl.ds(row0, _CH)],
                                osems[slot])

    def load_pos(j):
        return pltpu.async_copy(ptab_hbm.at[pl.ds(s0 + j * _CH, _CH)],
                                pbufs[j % 2], psems[j % 2])

    p_descs = [None] * _NG
    g_descs = [[None] * _BATCH for _ in range(_NG)]
    o_descs = [[None] * _BATCH for _ in range(_NG)]
    p_descs[0] = load_pos(0)
    for b in range(_BATCH):
        g_descs[0][b] = gather(0, b)

    for j in range(_NG):
        half = j % 2
        if j + 1 < _NG:
            p_descs[j + 1] = load_pos(j + 1)
            if j >= 1:
                for b in range(_BATCH):
                    o_descs[j - 1][b].wait()
            for b in range(_BATCH):
                g_descs[j + 1][b] = gather(j + 1, b)
        p_descs[j].wait()
        for b in range(_BATCH):
            g_descs[j][b].wait()
        wset = wbufs[half * _BATCH:(half + 1) * _BATCH]
        pb = pbufs[half]

        def blk(i, _, wset=wset, pb=pb):
            r = i // 16
            g0 = (i % 16) * 4
            for g in range(4):
                sl = pl.ds((g0 + g) * _LANES, _LANES)
                x = pb[r, sl]
                for wb in wset:
                    plsc.addupdate(wb.at[r, sl], x)
            return 0

        lax.fori_loop(0, _CH * 16, blk, 0)
        for b in range(_BATCH):
            o_descs[j][b] = writeback(j, b)

    for b in range(_BATCH):
        o_descs[_NG - 2][b].wait()
        o_descs[_NG - 1][b].wait()


def kernel(inputs, word_embeddings, position_embeddings):
    flat_idx = inputs.reshape(_TOK).astype(jnp.int32)
    out = _emb_kernel(flat_idx, word_embeddings, position_embeddings)
    return out.reshape(_BATCH, _SEQ, _HIDDEN)


# speedup vs baseline: 1.1884x; 1.0073x over previous
import functools

import jax
import jax.numpy as jnp
from jax import lax
from jax.experimental import pallas as pl
from jax.experimental.pallas import tpu as pltpu
from jax.experimental.pallas import tpu_sc as plsc

_HIDDEN = 1024
_BATCH = 4
_SEQ = 2048
_TOK = _BATCH * _SEQ
_NW = 32
_SPW = _SEQ // _NW
_CH = 8
_NG = _SPW // _CH
_LANES = 16
_HG = _HIDDEN // _LANES

_mesh = plsc.VectorSubcoreMesh(core_axis_name="c", subcore_axis_name="s")


@functools.partial(
    pl.kernel,
    out_type=jax.ShapeDtypeStruct((_TOK, _HIDDEN), jnp.float32),
    mesh=_mesh,
    scratch_types=[
        pltpu.VMEM((_BATCH * _SPW,), jnp.int32),
        [pltpu.VMEM((_CH, _HIDDEN), jnp.float32)] * (2 * _BATCH),
        [pltpu.VMEM((_CH, _HIDDEN), jnp.float32)] * 2,
        [pltpu.SemaphoreType.DMA] * (2 * _BATCH),
        [pltpu.SemaphoreType.DMA] * (2 * _BATCH),
        [pltpu.SemaphoreType.DMA] * 2,
    ],
)
def _emb_kernel(idx_hbm, wtab_hbm, ptab_hbm, out_hbm, idx_v, wbufs, pbufs,
                gsems, osems, psems):
    wid = lax.axis_index("s") * 2 + lax.axis_index("c")
    s0 = wid * _SPW

    idx_descs = [
        pltpu.async_copy(idx_hbm.at[pl.ds(b * _SEQ + s0, _SPW)],
                         idx_v.at[pl.ds(b * _SPW, _SPW)], osems[b])
        for b in range(_BATCH)
    ]
    for d in idx_descs:
        d.wait()

    def gather(j, b):
        slot = (j % 2) * _BATCH + b
        return pltpu.async_copy(
            wtab_hbm.at[idx_v.at[pl.ds(b * _SPW + j * _CH, _CH)]],
            wbufs[slot], gsems[slot])

    def writeback(j, b):
        slot = (j % 2) * _BATCH + b
        row0 = b * _SEQ + s0 + j * _CH
        return pltpu.async_copy(wbufs[slot], out_hbm.at[pl.ds(row0, _CH)],
                                osems[slot])

    def load_pos(j):
        return pltpu.async_copy(ptab_hbm.at[pl.ds(s0 + j * _CH, _CH)],
                                pbufs[j % 2], psems[j % 2])

    p_descs = [None] * _NG
    g_descs = [[None] * _BATCH for _ in range(_NG)]
    o_descs = [[None] * _BATCH for _ in range(_NG)]
    p_descs[0] = load_pos(0)
    for b in range(_BATCH):
        g_descs[0][b] = gather(0, b)

    for j in range(_NG):
        half = j % 2
        if j + 1 < _NG:
            p_descs[j + 1] = load_pos(j + 1)
            if j >= 1:
                for b in range(_BATCH):
                    o_descs[j - 1][b].wait()
            for b in range(_BATCH):
                g_descs[j + 1][b] = gather(j + 1, b)
        p_descs[j].wait()
        for b in range(_BATCH):
            g_descs[j][b].wait()
        wset = wbufs[half * _BATCH:(half + 1) * _BATCH]
        pb = pbufs[half]

        def blk(i, _, wset=wset, pb=pb):
            r = i // 32
            g0 = (i % 32) * 2
            for g in range(2):
                sl = pl.ds((g0 + g) * _LANES, _LANES)
                x = pb[r, sl]
                for wb in wset:
                    plsc.addupdate(wb.at[r, sl], x)
            return 0

        lax.fori_loop(0, _CH * 32, blk, 0)
        for b in range(_BATCH):
            o_descs[j][b] = writeback(j, b)

    for b in range(_BATCH):
        o_descs[_NG - 2][b].wait()
        o_descs[_NG - 1][b].wait()


def kernel(inputs, word_embeddings, position_embeddings):
    flat_idx = inputs.reshape(_TOK).astype(jnp.int32)
    out = _emb_kernel(flat_idx, word_embeddings, position_embeddings)
    return out.reshape(_BATCH, _SEQ, _HIDDEN)
